# Initial kernel scaffold; baseline (speedup 1.0000x reference)
import functools

import jax
import jax.numpy as jnp
from jax import lax
from jax.experimental import pallas as pl
from jax.experimental.pallas import tpu as pltpu
from jax.experimental.pallas import tpu_sc as plsc

N = 10000
E = 320000
IN_DIM = 128
HID1 = 128
HID2 = 64
N_CLASSES = 32

NC = 2
NS = 16
NW = NC * NS
CHUNK = 128
NCH = 79
E_PAD = NW * NCH * CHUNK
DUMMY = N
NPAD = 10016
RPS = NPAD // NS

_mesh = plsc.VectorSubcoreMesh(core_axis_name="c", subcore_axis_name="s")


@functools.partial(
    pl.kernel,
    out_type=jax.ShapeDtypeStruct((NC * NPAD, 16), jnp.float32),
    mesh=_mesh,
    scratch_types=[
        pltpu.VMEM((NCH, CHUNK), jnp.int32),
        pltpu.VMEM((CHUNK, 16), jnp.float32),
        pltpu.VMEM_SHARED((NPAD, 16), jnp.float32),
    ],
)
def _sc_degree(dst_hbm, ones_hbm, zeros_hbm, out_hbm, idx_v, ones_v, acc):
    c = lax.axis_index("c")
    s = lax.axis_index("s")
    w = c * NS + s
    row0 = s * RPS
    pltpu.sync_copy(zeros_hbm.at[pl.ds(row0, RPS)], acc.at[pl.ds(row0, RPS)])
    pltpu.sync_copy(dst_hbm.at[w], idx_v)
    pltpu.sync_copy(ones_hbm, ones_v)
    plsc.subcore_barrier()

    @pl.loop(0, NCH)
    def _(j):
        pltpu.sync_copy(ones_v, acc.at[idx_v.at[j]], add=True)

    plsc.subcore_barrier()
    pltpu.sync_copy(acc.at[pl.ds(row0, RPS)],
                    out_hbm.at[pl.ds(c * NPAD + row0, RPS)])


def _make_sc_aggregate(d):
    @functools.partial(
        pl.kernel,
        out_type=jax.ShapeDtypeStruct((NC * NPAD, d), jnp.float32),
        mesh=_mesh,
        scratch_types=[
            pltpu.VMEM((NCH, CHUNK), jnp.int32),
            pltpu.VMEM((NCH, CHUNK), jnp.int32),
            pltpu.VMEM((CHUNK, d), jnp.float32),
            pltpu.VMEM_SHARED((NPAD, d), jnp.float32),
        ],
    )
    def _sc_aggregate(h_hbm, src_hbm, dst_hbm, zeros_hbm, out_hbm,
                      src_v, dst_v, buf, acc):
        c = lax.axis_index("c")
        s = lax.axis_index("s")
        w = c * NS + s
        row0 = s * RPS
        pltpu.sync_copy(zeros_hbm.at[pl.ds(row0, RPS)],
                        acc.at[pl.ds(row0, RPS)])
        pltpu.sync_copy(src_hbm.at[w], src_v)
        pltpu.sync_copy(dst_hbm.at[w], dst_v)
        plsc.subcore_barrier()

        @pl.loop(0, NCH)
        def _(j):
            pltpu.sync_copy(h_hbm.at[src_v.at[j]], buf)
            pltpu.sync_copy(buf, acc.at[dst_v.at[j]], add=True)

        plsc.subcore_barrier()
        pltpu.sync_copy(acc.at[pl.ds(row0, RPS)],
                        out_hbm.at[pl.ds(c * NPAD + row0, RPS)])

    return _sc_aggregate


_sc_aggregate_h1 = _make_sc_aggregate(HID1)
_sc_aggregate_h2 = _make_sc_aggregate(HID2)


def _tc_matmul(x, w):
    def body(x_ref, w_ref, o_ref):
        o_ref[...] = jnp.dot(x_ref[...], w_ref[...],
                             preferred_element_type=jnp.float32)

    return pl.pallas_call(
        body,
        out_shape=jax.ShapeDtypeStruct((x.shape[0], w.shape[1]), jnp.float32),
    )(x, w)


def _tc_norm(degp, h1):
    def body(dp_ref, h_ref, dis_ref, hs_ref):
        deg = dp_ref[0, :, 0:1] + dp_ref[1, :, 0:1] + 1.0
        dis = lax.rsqrt(deg)
        dis_ref[...] = dis
        hs_ref[...] = h_ref[...] * dis

    return pl.pallas_call(
        body,
        out_shape=(
            jax.ShapeDtypeStruct((NPAD, 1), jnp.float32),
            jax.ShapeDtypeStruct((NPAD, IN_DIM), jnp.float32),
        ),
    )(degp, h1)


def _tc_layer(agg, h, dis, b, w):
    def body(a_ref, h_ref, dis_ref, b_ref, w_ref, h2_ref, h2s_ref):
        dis = dis_ref[...]
        pre = ((a_ref[0] + a_ref[1]) * dis
               + h_ref[...] * (dis * dis) + b_ref[...])
        o1 = jnp.maximum(pre, 0.0)
        h2 = jnp.dot(o1, w_ref[...], preferred_element_type=jnp.float32)
        h2_ref[...] = h2
        h2s_ref[...] = h2 * dis

    d2 = w.shape[1]
    return pl.pallas_call(
        body,
        out_shape=(
            jax.ShapeDtypeStruct((NPAD, d2), jnp.float32),
            jax.ShapeDtypeStruct((NPAD, d2), jnp.float32),
        ),
    )(agg, h, dis, b, w)


def _tc_final(agg, h, dis, b, wf, bf):
    def body(a_ref, h_ref, dis_ref, b_ref, w_ref, bf_ref, o_ref):
        dis = dis_ref[...]
        pre = ((a_ref[0] + a_ref[1]) * dis
               + h_ref[...] * (dis * dis) + b_ref[...])
        o2 = jnp.maximum(pre, 0.0)
        o_ref[...] = jnp.dot(o2, w_ref[...],
                             preferred_element_type=jnp.float32) + bf_ref[...]

    return pl.pallas_call(
        body,
        out_shape=jax.ShapeDtypeStruct((NPAD, N_CLASSES), jnp.float32),
    )(agg, h, dis, b, wf, bf)


def kernel(x, edge_index, W1, b1, W2, b2, Wf, bf):
    src = edge_index[0].astype(jnp.int32)
    dst = edge_index[1].astype(jnp.int32)
    pad = jnp.full((E_PAD - E,), DUMMY, jnp.int32)
    src_r = jnp.concatenate([src, pad]).reshape(NW, NCH, CHUNK)
    dst_r = jnp.concatenate([dst, pad]).reshape(NW, NCH, CHUNK)
    x_pad = jnp.concatenate(
        [x, jnp.zeros((NPAD - N, IN_DIM), jnp.float32)], axis=0)

    ones16 = jnp.ones((CHUNK, 16), jnp.float32)
    zeros16 = jnp.zeros((NPAD, 16), jnp.float32)
    zeros1 = jnp.zeros((NPAD, HID1), jnp.float32)
    zeros2 = jnp.zeros((NPAD, HID2), jnp.float32)

    degp = _sc_degree(dst_r, ones16, zeros16).reshape(NC, NPAD, 16)
    h1 = _tc_matmul(x_pad, W1)

    dis, h1s = _tc_norm(degp, h1)

    agg1 = _sc_aggregate_h1(h1s, src_r, dst_r, zeros1).reshape(NC, NPAD, HID1)
    h2, h2s = _tc_layer(agg1, h1, dis, b1.reshape(1, HID1), W2)

    agg2 = _sc_aggregate_h2(h2s, src_r, dst_r, zeros2).reshape(NC, NPAD, HID2)
    out = _tc_final(agg2, h2, dis, b2.reshape(1, HID2), Wf,
                    bf.reshape(1, N_CLASSES))
    return out[:N]


# speedup vs baseline: 15.4379x; 15.4379x over previous
import functools

import jax
import jax.numpy as jnp
from jax import lax
from jax.experimental import pallas as pl
from jax.experimental.pallas import tpu as pltpu
from jax.experimental.pallas import tpu_sc as plsc

N = 10000
E = 320000
IN_DIM = 128
HID1 = 128
HID2 = 64
N_CLASSES = 32

NC = 2
NS = 16
NW = NC * NS
CHUNK = 128
NCH = 79
E_PAD = NW * NCH * CHUNK
DUMMY = N
NPAD = 10112
RPS = NPAD // NS

_mesh = plsc.VectorSubcoreMesh(core_axis_name="c", subcore_axis_name="s")


@functools.partial(
    pl.kernel,
    out_type=jax.ShapeDtypeStruct((NC * NPAD, 16), jnp.float32),
    mesh=_mesh,
    compiler_params=pltpu.CompilerParams(use_tc_tiling_on_sc=False),
    scratch_types=[
        pltpu.VMEM((NCH, CHUNK), jnp.int32),
        pltpu.VMEM((CHUNK, 16), jnp.float32),
        pltpu.VMEM_SHARED((NPAD, 16), jnp.float32),
    ],
)
def _sc_degree(dst_hbm, ones_hbm, zeros_hbm, out_hbm, idx_v, ones_v, acc):
    c = lax.axis_index("c")
    s = lax.axis_index("s")
    w = c * NS + s
    row0 = s * RPS
    pltpu.sync_copy(zeros_hbm.at[pl.ds(row0, RPS)], acc.at[pl.ds(row0, RPS)])
    pltpu.sync_copy(dst_hbm.at[w], idx_v)
    pltpu.sync_copy(ones_hbm, ones_v)
    plsc.subcore_barrier()

    @pl.loop(0, NCH)
    def _(j):
        pltpu.sync_copy(ones_v, acc.at[idx_v.at[j]], add=True)

    plsc.subcore_barrier()
    pltpu.sync_copy(acc.at[pl.ds(row0, RPS)],
                    out_hbm.at[pl.ds(c * NPAD + row0, RPS)])


def _make_sc_aggregate(d):
    @functools.partial(
        pl.kernel,
        out_type=jax.ShapeDtypeStruct((NC * NPAD, d), jnp.float32),
        mesh=_mesh,
        compiler_params=pltpu.CompilerParams(use_tc_tiling_on_sc=False),
        scratch_types=[
            pltpu.VMEM((NCH, CHUNK), jnp.int32),
            pltpu.VMEM((NCH, CHUNK), jnp.int32),
            pltpu.VMEM((CHUNK, d), jnp.float32),
            pltpu.VMEM_SHARED((NPAD, d), jnp.float32),
        ],
    )
    def _sc_aggregate(h_hbm, src_hbm, dst_hbm, zeros_hbm, out_hbm,
                      src_v, dst_v, buf, acc):
        c = lax.axis_index("c")
        s = lax.axis_index("s")
        w = c * NS + s
        row0 = s * RPS
        pltpu.sync_copy(zeros_hbm.at[pl.ds(row0, RPS)],
                        acc.at[pl.ds(row0, RPS)])
        pltpu.sync_copy(src_hbm.at[w], src_v)
        pltpu.sync_copy(dst_hbm.at[w], dst_v)
        plsc.subcore_barrier()

        @pl.loop(0, NCH)
        def _(j):
            pltpu.sync_copy(h_hbm.at[src_v.at[j]], buf)
            pltpu.sync_copy(buf, acc.at[dst_v.at[j]], add=True)

        plsc.subcore_barrier()
        pltpu.sync_copy(acc.at[pl.ds(row0, RPS)],
                        out_hbm.at[pl.ds(c * NPAD + row0, RPS)])

    return _sc_aggregate


_sc_aggregate_h1 = _make_sc_aggregate(HID1)
_sc_aggregate_h2 = _make_sc_aggregate(HID2)


def _tc_matmul(x, w):
    def body(x_ref, w_ref, o_ref):
        o_ref[...] = jnp.dot(x_ref[...], w_ref[...],
                             preferred_element_type=jnp.float32)

    return pl.pallas_call(
        body,
        out_shape=jax.ShapeDtypeStruct((x.shape[0], w.shape[1]), jnp.float32),
    )(x, w)


def _tc_norm(degp, h1):
    def body(dp_ref, h_ref, dis_ref, hs_ref):
        deg = dp_ref[0, :, 0:1] + dp_ref[1, :, 0:1] + 1.0
        dis = lax.rsqrt(deg)
        dis_ref[...] = dis
        hs_ref[...] = h_ref[...] * dis

    return pl.pallas_call(
        body,
        out_shape=(
            jax.ShapeDtypeStruct((NPAD, 1), jnp.float32),
            jax.ShapeDtypeStruct((NPAD, IN_DIM), jnp.float32),
        ),
    )(degp, h1)


def _tc_layer(agg, h, dis, b, w):
    def body(a_ref, h_ref, dis_ref, b_ref, w_ref, h2_ref, h2s_ref):
        dis = dis_ref[...]
        pre = ((a_ref[0] + a_ref[1]) * dis
               + h_ref[...] * (dis * dis) + b_ref[...])
        o1 = jnp.maximum(pre, 0.0)
        h2 = jnp.dot(o1, w_ref[...], preferred_element_type=jnp.float32)
        h2_ref[...] = h2
        h2s_ref[...] = h2 * dis

    d2 = w.shape[1]
    return pl.pallas_call(
        body,
        out_shape=(
            jax.ShapeDtypeStruct((NPAD, d2), jnp.float32),
            jax.ShapeDtypeStruct((NPAD, d2), jnp.float32),
        ),
    )(agg, h, dis, b, w)


def _tc_final(agg, h, dis, b, wf, bf):
    def body(a_ref, h_ref, dis_ref, b_ref, w_ref, bf_ref, o_ref):
        dis = dis_ref[...]
        pre = ((a_ref[0] + a_ref[1]) * dis
               + h_ref[...] * (dis * dis) + b_ref[...])
        o2 = jnp.maximum(pre, 0.0)
        o_ref[...] = jnp.dot(o2, w_ref[...],
                             preferred_element_type=jnp.float32) + bf_ref[...]

    return pl.pallas_call(
        body,
        out_shape=jax.ShapeDtypeStruct((NPAD, N_CLASSES), jnp.float32),
    )(agg, h, dis, b, wf, bf)


def kernel(x, edge_index, W1, b1, W2, b2, Wf, bf):
    src = edge_index[0].astype(jnp.int32)
    dst = edge_index[1].astype(jnp.int32)
    pad = jnp.full((E_PAD - E,), DUMMY, jnp.int32)
    src_r = jnp.concatenate([src, pad]).reshape(NW, NCH, CHUNK)
    dst_r = jnp.concatenate([dst, pad]).reshape(NW, NCH, CHUNK)
    x_pad = jnp.concatenate(
        [x, jnp.zeros((NPAD - N, IN_DIM), jnp.float32)], axis=0)

    ones16 = jnp.ones((CHUNK, 16), jnp.float32)
    zeros16 = jnp.zeros((NPAD, 16), jnp.float32)
    zeros1 = jnp.zeros((NPAD, HID1), jnp.float32)
    zeros2 = jnp.zeros((NPAD, HID2), jnp.float32)

    degp = _sc_degree(dst_r, ones16, zeros16).reshape(NC, NPAD, 16)
    h1 = _tc_matmul(x_pad, W1)

    dis, h1s = _tc_norm(degp, h1)

    agg1 = _sc_aggregate_h1(h1s, src_r, dst_r, zeros1).reshape(NC, NPAD, HID1)
    h2, h2s = _tc_layer(agg1, h1, dis, b1.reshape(1, HID1), W2)

    agg2 = _sc_aggregate_h2(h2s, src_r, dst_r, zeros2).reshape(NC, NPAD, HID2)
    out = _tc_final(agg2, h2, dis, b2.reshape(1, HID2), Wf,
                    bf.reshape(1, N_CLASSES))
    return out[:N]


# speedup vs baseline: 32.4214x; 2.1001x over previous
import functools

import jax
import jax.numpy as jnp
from jax import lax
from jax.experimental import pallas as pl
from jax.experimental.pallas import tpu as pltpu
from jax.experimental.pallas import tpu_sc as plsc

N = 10000
E = 320000
IN_DIM = 128
HID1 = 128
HID2 = 64
N_CLASSES = 32

NC = 2
NS = 16
NW = NC * NS
CHUNK = 128
NBUF = 4
E_PAD = 327680
NCH1 = E_PAD // (NS * CHUNK)
NCH2 = E_PAD // (NW * CHUNK)
DUMMY = N
NPAD = 10112
RPS = NPAD // NS

_mesh = plsc.VectorSubcoreMesh(core_axis_name="c", subcore_axis_name="s")
_sc_params = pltpu.CompilerParams(use_tc_tiling_on_sc=False)


@functools.partial(
    pl.kernel,
    out_type=jax.ShapeDtypeStruct((NC * NPAD, 16), jnp.float32),
    mesh=_mesh,
    compiler_params=_sc_params,
    scratch_types=[
        pltpu.VMEM((NCH2, CHUNK), jnp.int32),
        pltpu.VMEM((CHUNK, 16), jnp.float32),
        pltpu.VMEM_SHARED((NPAD, 16), jnp.float32),
    ],
)
def _sc_degree(dst_hbm, ones_hbm, zeros_hbm, out_hbm, idx_v, ones_v, acc):
    c = lax.axis_index("c")
    s = lax.axis_index("s")
    w = c * NS + s
    row0 = s * RPS
    pltpu.sync_copy(zeros_hbm.at[pl.ds(row0, RPS)], acc.at[pl.ds(row0, RPS)])
    pltpu.sync_copy(dst_hbm.at[w], idx_v)
    pltpu.sync_copy(ones_hbm, ones_v)
    plsc.subcore_barrier()

    @pl.loop(0, NCH2)
    def _(j):
        pltpu.sync_copy(ones_v, acc.at[idx_v.at[j]], add=True)

    plsc.subcore_barrier()
    pltpu.sync_copy(acc.at[pl.ds(row0, RPS)],
                    out_hbm.at[pl.ds(c * NPAD + row0, RPS)])


def _make_sc_aggregate(nch, rows_in):
    @functools.partial(
        pl.kernel,
        out_type=jax.ShapeDtypeStruct((NC * NPAD, HID2), jnp.float32),
        mesh=_mesh,
        compiler_params=_sc_params,
        scratch_types=[
            pltpu.VMEM((nch, CHUNK), jnp.int32),
            pltpu.VMEM((nch, CHUNK), jnp.int32),
            [pltpu.VMEM((CHUNK, HID2), jnp.float32)] * NBUF,
            [pltpu.SemaphoreType.DMA] * NBUF,
            [pltpu.SemaphoreType.DMA] * NBUF,
            pltpu.VMEM_SHARED((NPAD, HID2), jnp.float32),
        ],
    )
    def _sc_aggregate(h_hbm, src_hbm, dst_hbm, zeros_hbm, out_hbm,
                      src_v, dst_v, bufs, gsems, ssems, acc):
        c = lax.axis_index("c")
        s = lax.axis_index("s")
        w = c * NS + s
        row0 = s * RPS
        pltpu.sync_copy(src_hbm.at[w], src_v)
        pltpu.sync_copy(dst_hbm.at[w], dst_v)
        pltpu.sync_copy(zeros_hbm.at[pl.ds(row0, RPS)],
                        acc.at[pl.ds(row0, RPS)])
        for b in range(NBUF):
            pltpu.async_copy(h_hbm.at[src_v.at[b]], bufs[b], gsems[b])
        plsc.subcore_barrier()

        @pl.loop(0, nch - NBUF, step=NBUF)
        def _(j):
            for b in range(NBUF):
                pltpu.make_async_copy(
                    h_hbm.at[src_v.at[j + b]], bufs[b], gsems[b]).wait()
                pltpu.async_copy(
                    bufs[b], acc.at[dst_v.at[j + b]], ssems[b], add=True)
            for b in range(NBUF):
                pltpu.make_async_copy(
                    bufs[b], acc.at[dst_v.at[j + b]], ssems[b]).wait()
                pltpu.async_copy(
                    h_hbm.at[src_v.at[j + b + NBUF]], bufs[b], gsems[b])

        for b in range(NBUF):
            j = nch - NBUF + b
            pltpu.make_async_copy(
                h_hbm.at[src_v.at[j]], bufs[b], gsems[b]).wait()
            pltpu.async_copy(
                bufs[b], acc.at[dst_v.at[j]], ssems[b], add=True)
        for b in range(NBUF):
            j = nch - NBUF + b
            pltpu.make_async_copy(
                bufs[b], acc.at[dst_v.at[j]], ssems[b]).wait()

        plsc.subcore_barrier()
        pltpu.sync_copy(acc.at[pl.ds(row0, RPS)],
                        out_hbm.at[pl.ds(c * NPAD + row0, RPS)])

    return _sc_aggregate


_sc_agg_l1 = _make_sc_aggregate(NCH1, NC * NPAD)
_sc_agg_l2 = _make_sc_aggregate(NCH2, NPAD)


def _tc_matmul(x, w):
    def body(x_ref, w_ref, o_ref):
        o_ref[...] = jnp.dot(x_ref[...], w_ref[...],
                             preferred_element_type=jnp.float32)

    return pl.pallas_call(
        body,
        out_shape=jax.ShapeDtypeStruct((x.shape[0], w.shape[1]), jnp.float32),
    )(x, w)


def _tc_norm(degp, h1):
    def body(dp_ref, h_ref, dis_ref, hs_ref):
        deg = dp_ref[0, :, 0:1] + dp_ref[1, :, 0:1] + 1.0
        dis = lax.rsqrt(deg)
        dis_ref[...] = dis
        hs = h_ref[...] * dis
        hs_ref[0] = hs[:, :HID2]
        hs_ref[1] = hs[:, HID2:]

    return pl.pallas_call(
        body,
        out_shape=(
            jax.ShapeDtypeStruct((NPAD, 1), jnp.float32),
            jax.ShapeDtypeStruct((NC, NPAD, HID2), jnp.float32),
        ),
    )(degp, h1)


def _tc_layer(agg, h, dis, b, w):
    def body(a_ref, h_ref, dis_ref, b_ref, w_ref, h2_ref, h2s_ref):
        dis = dis_ref[...]
        agg = jnp.concatenate([a_ref[0], a_ref[1]], axis=1)
        pre = agg * dis + h_ref[...] * (dis * dis) + b_ref[...]
        o1 = jnp.maximum(pre, 0.0)
        h2 = jnp.dot(o1, w_ref[...], preferred_element_type=jnp.float32)
        h2_ref[...] = h2
        h2s_ref[...] = h2 * dis

    d2 = w.shape[1]
    return pl.pallas_call(
        body,
        out_shape=(
            jax.ShapeDtypeStruct((NPAD, d2), jnp.float32),
            jax.ShapeDtypeStruct((NPAD, d2), jnp.float32),
        ),
    )(agg, h, dis, b, w)


def _tc_final(agg, h, dis, b, wf, bf):
    def body(a_ref, h_ref, dis_ref, b_ref, w_ref, bf_ref, o_ref):
        dis = dis_ref[...]
        pre = ((a_ref[0] + a_ref[1]) * dis
               + h_ref[...] * (dis * dis) + b_ref[...])
        o2 = jnp.maximum(pre, 0.0)
        o_ref[...] = jnp.dot(o2, w_ref[...],
                             preferred_element_type=jnp.float32) + bf_ref[...]

    return pl.pallas_call(
        body,
        out_shape=jax.ShapeDtypeStruct((NPAD, N_CLASSES), jnp.float32),
    )(agg, h, dis, b, wf, bf)


def kernel(x, edge_index, W1, b1, W2, b2, Wf, bf):
    src = edge_index[0].astype(jnp.int32)
    dst = edge_index[1].astype(jnp.int32)
    pad = DUMMY + jnp.arange(E_PAD - E, dtype=jnp.int32) % (NPAD - N)
    src_p = jnp.concatenate([src, pad])
    dst_p = jnp.concatenate([dst, pad])

    src16 = src_p.reshape(NS, NCH1, CHUNK)
    dst16 = dst_p.reshape(NS, NCH1, CHUNK)
    src_l1 = jnp.concatenate([src16, src16 + NPAD], axis=0)
    dst_l1 = jnp.concatenate([dst16, dst16], axis=0)

    src_l2 = src_p.reshape(NW, NCH2, CHUNK)
    dst_l2 = dst_p.reshape(NW, NCH2, CHUNK)

    x_pad = jnp.concatenate(
        [x, jnp.zeros((NPAD - N, IN_DIM), jnp.float32)], axis=0)

    ones16 = jnp.ones((CHUNK, 16), jnp.float32)
    zeros16 = jnp.zeros((NPAD, 16), jnp.float32)
    zeros64 = jnp.zeros((NPAD, HID2), jnp.float32)

    degp = _sc_degree(dst_l2, ones16, zeros16).reshape(NC, NPAD, 16)
    h1 = _tc_matmul(x_pad, W1)

    dis, h1s = _tc_norm(degp, h1)
    h1s_cat = h1s.reshape(NC * NPAD, HID2)

    agg1 = _sc_agg_l1(h1s_cat, src_l1, dst_l1, zeros64).reshape(
        NC, NPAD, HID2)
    h2, h2s = _tc_layer(agg1, h1, dis, b1.reshape(1, HID1), W2)

    agg2 = _sc_agg_l2(h2s, src_l2, dst_l2, zeros64).reshape(NC, NPAD, HID2)
    out = _tc_final(agg2, h2, dis, b2.reshape(1, HID2), Wf,
                    bf.reshape(1, N_CLASSES))
    return out[:N]


# speedup vs baseline: 33.2118x; 1.0244x over previous
import functools

import jax
import jax.numpy as jnp
from jax import lax
from jax.experimental import pallas as pl
from jax.experimental.pallas import tpu as pltpu
from jax.experimental.pallas import tpu_sc as plsc

N = 10000
E = 320000
IN_DIM = 128
HID1 = 128
HID2 = 64
N_CLASSES = 32

NC = 2
NS = 16
NW = NC * NS
CHUNK = 128
NBUF = 4
E_PAD = 327680
NCH1 = E_PAD // (NS * CHUNK)
NCH2 = E_PAD // (NW * CHUNK)
DUMMY = N
NPAD = 10112
RPS = NPAD // NS

_mesh = plsc.VectorSubcoreMesh(core_axis_name="c", subcore_axis_name="s")
_sc_params = pltpu.CompilerParams(use_tc_tiling_on_sc=False)


def _zero_rows(zbuf, acc, row0, width):
    del width
    for q in range(RPS // CHUNK):
        pltpu.sync_copy(zbuf, acc.at[pl.ds(row0 + q * CHUNK, CHUNK)])
    rem = RPS - (RPS // CHUNK) * CHUNK
    if rem:
        pltpu.sync_copy(zbuf.at[pl.ds(0, rem)],
                        acc.at[pl.ds(row0 + (RPS // CHUNK) * CHUNK, rem)])


@functools.partial(
    pl.kernel,
    out_type=jax.ShapeDtypeStruct((NC * NPAD, 16), jnp.float32),
    mesh=_mesh,
    compiler_params=_sc_params,
    scratch_types=[
        pltpu.VMEM((NCH2, CHUNK), jnp.int32),
        pltpu.VMEM((CHUNK, 16), jnp.float32),
        pltpu.VMEM((CHUNK, 16), jnp.float32),
        pltpu.VMEM_SHARED((NPAD, 16), jnp.float32),
    ],
)
def _sc_degree(dst_hbm, out_hbm, idx_v, ones_v, zbuf, acc):
    c = lax.axis_index("c")
    s = lax.axis_index("s")
    w = c * NS + s
    row0 = s * RPS
    pltpu.sync_copy(dst_hbm.at[w], idx_v)

    @pl.loop(0, CHUNK)
    def _(r):
        ones_v[r, pl.ds(0, 16)] = jnp.full((16,), 1.0, jnp.float32)
        zbuf[r, pl.ds(0, 16)] = jnp.zeros((16,), jnp.float32)

    _zero_rows(zbuf, acc, row0, 16)
    plsc.subcore_barrier()

    @pl.loop(0, NCH2)
    def _(j):
        pltpu.sync_copy(ones_v, acc.at[idx_v.at[j]], add=True)

    plsc.subcore_barrier()
    pltpu.sync_copy(acc.at[pl.ds(row0, RPS)],
                    out_hbm.at[pl.ds(c * NPAD + row0, RPS)])


def _make_sc_aggregate(nch, rows_in, dst_by_s):
    @functools.partial(
        pl.kernel,
        out_type=jax.ShapeDtypeStruct((NC * NPAD, HID2), jnp.float32),
        mesh=_mesh,
        compiler_params=_sc_params,
        scratch_types=[
            pltpu.VMEM((nch, CHUNK), jnp.int32),
            pltpu.VMEM((nch, CHUNK), jnp.int32),
            [pltpu.VMEM((CHUNK, HID2), jnp.float32)] * NBUF,
            pltpu.VMEM((CHUNK, HID2), jnp.float32),
            [pltpu.SemaphoreType.DMA] * NBUF,
            [pltpu.SemaphoreType.DMA] * NBUF,
            pltpu.VMEM_SHARED((NPAD, HID2), jnp.float32),
        ],
    )
    def _sc_aggregate(h_hbm, src_hbm, dst_hbm, out_hbm,
                      src_v, dst_v, bufs, zbuf, gsems, ssems, acc):
        c = lax.axis_index("c")
        s = lax.axis_index("s")
        w = c * NS + s
        row0 = s * RPS
        pltpu.sync_copy(src_hbm.at[w], src_v)
        pltpu.sync_copy(dst_hbm.at[s if dst_by_s else w], dst_v)
        for b in range(NBUF):
            pltpu.async_copy(h_hbm.at[src_v.at[b]], bufs[b], gsems[b])

        @pl.loop(0, CHUNK)
        def _(r):
            for k in range(0, HID2, 16):
                zbuf[r, pl.ds(k, 16)] = jnp.zeros((16,), jnp.float32)

        _zero_rows(zbuf, acc, row0, HID2)
        plsc.subcore_barrier()

        @pl.loop(0, nch - NBUF, step=NBUF)
        def _(j):
            for b in range(NBUF):
                pltpu.make_async_copy(
                    h_hbm.at[src_v.at[j + b]], bufs[b], gsems[b]).wait()
                pltpu.async_copy(
                    bufs[b], acc.at[dst_v.at[j + b]], ssems[b], add=True)
            for b in range(NBUF):
                pltpu.make_async_copy(
                    bufs[b], acc.at[dst_v.at[j + b]], ssems[b]).wait()
                pltpu.async_copy(
                    h_hbm.at[src_v.at[j + b + NBUF]], bufs[b], gsems[b])

        for b in range(NBUF):
            j = nch - NBUF + b
            pltpu.make_async_copy(
                h_hbm.at[src_v.at[j]], bufs[b], gsems[b]).wait()
            pltpu.async_copy(
                bufs[b], acc.at[dst_v.at[j]], ssems[b], add=True)
        for b in range(NBUF):
            j = nch - NBUF + b
            pltpu.make_async_copy(
                bufs[b], acc.at[dst_v.at[j]], ssems[b]).wait()

        plsc.subcore_barrier()
        pltpu.sync_copy(acc.at[pl.ds(row0, RPS)],
                        out_hbm.at[pl.ds(c * NPAD + row0, RPS)])

    return _sc_aggregate


_sc_agg_l1 = _make_sc_aggregate(NCH1, NC * NPAD, True)
_sc_agg_l2 = _make_sc_aggregate(NCH2, NPAD, False)


def _tc_mm_norm(x, w, degp):
    def body(x_ref, w_ref, dp_ref, h1_ref, dis_ref, hs_ref):
        h1 = jnp.dot(x_ref[...], w_ref[...],
                     preferred_element_type=jnp.float32)
        deg = dp_ref[0, :, 0:1] + dp_ref[1, :, 0:1] + 1.0
        dis = lax.rsqrt(deg)
        dis_ref[...] = dis
        zpad = jnp.zeros((NPAD - N, IN_DIM), jnp.float32)
        h1_ref[pl.ds(0, N), :] = h1
        h1_ref[pl.ds(N, NPAD - N), :] = zpad
        hs = h1 * dis[:N]
        hs_ref[0, pl.ds(0, N), :] = hs[:, :HID2]
        hs_ref[0, pl.ds(N, NPAD - N), :] = zpad[:, :HID2]
        hs_ref[1, pl.ds(0, N), :] = hs[:, HID2:]
        hs_ref[1, pl.ds(N, NPAD - N), :] = zpad[:, :HID2]

    return pl.pallas_call(
        body,
        out_shape=(
            jax.ShapeDtypeStruct((NPAD, IN_DIM), jnp.float32),
            jax.ShapeDtypeStruct((NPAD, 1), jnp.float32),
            jax.ShapeDtypeStruct((NC, NPAD, HID2), jnp.float32),
        ),
    )(x, w, degp)


def _tc_layer(agg, h, dis, b, w):
    def body(a_ref, h_ref, dis_ref, b_ref, w_ref, h2_ref, h2s_ref):
        dis = dis_ref[...]
        agg = jnp.concatenate([a_ref[0], a_ref[1]], axis=1)
        pre = agg * dis + h_ref[...] * (dis * dis) + b_ref[...]
        o1 = jnp.maximum(pre, 0.0)
        h2 = jnp.dot(o1, w_ref[...], preferred_element_type=jnp.float32)
        h2_ref[...] = h2
        h2s_ref[...] = h2 * dis

    d2 = w.shape[1]
    return pl.pallas_call(
        body,
        out_shape=(
            jax.ShapeDtypeStruct((NPAD, d2), jnp.float32),
            jax.ShapeDtypeStruct((NPAD, d2), jnp.float32),
        ),
    )(agg, h, dis, b, w)


def _tc_final(agg, h, dis, b, wf, bf):
    def body(a_ref, h_ref, dis_ref, b_ref, w_ref, bf_ref, o_ref):
        dis = dis_ref[...]
        pre = ((a_ref[0] + a_ref[1]) * dis
               + h_ref[...] * (dis * dis) + b_ref[...])
        o2 = jnp.maximum(pre, 0.0)
        o_ref[...] = jnp.dot(o2, w_ref[...],
                             preferred_element_type=jnp.float32) + bf_ref[...]

    return pl.pallas_call(
        body,
        out_shape=jax.ShapeDtypeStruct((NPAD, N_CLASSES), jnp.float32),
    )(agg, h, dis, b, wf, bf)


def kernel(x, edge_index, W1, b1, W2, b2, Wf, bf):
    src = edge_index[0].astype(jnp.int32)
    dst = edge_index[1].astype(jnp.int32)
    pad = DUMMY + jnp.arange(E_PAD - E, dtype=jnp.int32) % (NPAD - N)
    src_p = jnp.concatenate([src, pad])
    dst_p = jnp.concatenate([dst, pad])

    src16 = src_p.reshape(NS, NCH1, CHUNK)
    dst16 = dst_p.reshape(NS, NCH1, CHUNK)
    src_l1 = jnp.concatenate([src16, src16 + NPAD], axis=0)

    src_l2 = src_p.reshape(NW, NCH2, CHUNK)
    dst_l2 = dst_p.reshape(NW, NCH2, CHUNK)

    degp = _sc_degree(dst_l2).reshape(NC, NPAD, 16)
    h1, dis, h1s = _tc_mm_norm(x, W1, degp)
    h1s_cat = h1s.reshape(NC * NPAD, HID2)

    agg1 = _sc_agg_l1(h1s_cat, src_l1, dst16).reshape(NC, NPAD, HID2)
    h2, h2s = _tc_layer(agg1, h1, dis, b1.reshape(1, HID1), W2)

    agg2 = _sc_agg_l2(h2s, src_l2, dst_l2).reshape(NC, NPAD, HID2)
    out = _tc_final(agg2, h2, dis, b2.reshape(1, HID2), Wf,
                    bf.reshape(1, N_CLASSES))
    return out[:N]


# speedup vs baseline: 33.4326x; 1.0067x over previous
import functools

import jax
import jax.numpy as jnp
from jax import lax
from jax.experimental import pallas as pl
from jax.experimental.pallas import tpu as pltpu
from jax.experimental.pallas import tpu_sc as plsc

N = 10000
E = 320000
IN_DIM = 128
HID1 = 128
HID2 = 64
N_CLASSES = 32

NC = 2
NS = 16
NW = NC * NS
CHUNK = 128
NBUF = 4
E_PAD = 327680
NCH1 = E_PAD // (NS * CHUNK)
NCH2 = E_PAD // (NW * CHUNK)
DUMMY = N
NPAD = 10112
RPS = NPAD // NS

_mesh = plsc.VectorSubcoreMesh(core_axis_name="c", subcore_axis_name="s")
_sc_params = pltpu.CompilerParams(use_tc_tiling_on_sc=False)


def _zero_rows(zbuf, acc, row0, width):
    del width
    for q in range(RPS // CHUNK):
        pltpu.sync_copy(zbuf, acc.at[pl.ds(row0 + q * CHUNK, CHUNK)])
    rem = RPS - (RPS // CHUNK) * CHUNK
    if rem:
        pltpu.sync_copy(zbuf.at[pl.ds(0, rem)],
                        acc.at[pl.ds(row0 + (RPS // CHUNK) * CHUNK, rem)])


@functools.partial(
    pl.kernel,
    out_type=jax.ShapeDtypeStruct((NC * NPAD, 16), jnp.float32),
    mesh=_mesh,
    compiler_params=_sc_params,
    scratch_types=[
        pltpu.VMEM((NCH2, CHUNK), jnp.int32),
        pltpu.VMEM((CHUNK, 16), jnp.float32),
        pltpu.VMEM((CHUNK, 16), jnp.float32),
        pltpu.VMEM_SHARED((NPAD, 16), jnp.float32),
    ],
)
def _sc_degree(dst_hbm, out_hbm, idx_v, ones_v, zbuf, acc):
    c = lax.axis_index("c")
    s = lax.axis_index("s")
    w = c * NS + s
    row0 = s * RPS
    pltpu.sync_copy(dst_hbm.at[w], idx_v)

    @pl.loop(0, CHUNK)
    def _(r):
        ones_v[r, pl.ds(0, 16)] = jnp.full((16,), 1.0, jnp.float32)
        zbuf[r, pl.ds(0, 16)] = jnp.zeros((16,), jnp.float32)

    _zero_rows(zbuf, acc, row0, 16)
    plsc.subcore_barrier()

    @pl.loop(0, NCH2)
    def _(j):
        pltpu.sync_copy(ones_v, acc.at[idx_v.at[j]], add=True)

    plsc.subcore_barrier()
    pltpu.sync_copy(acc.at[pl.ds(row0, RPS)],
                    out_hbm.at[pl.ds(c * NPAD + row0, RPS)])


def _make_sc_aggregate(nch, rows_in, dst_by_s):
    @functools.partial(
        pl.kernel,
        out_type=jax.ShapeDtypeStruct((NC * NPAD, HID2), jnp.float32),
        mesh=_mesh,
        compiler_params=_sc_params,
        scratch_types=[
            pltpu.VMEM((nch, CHUNK), jnp.int32),
            pltpu.VMEM((nch, CHUNK), jnp.int32),
            [pltpu.VMEM((CHUNK, HID2), jnp.float32)] * NBUF,
            pltpu.VMEM((CHUNK, HID2), jnp.float32),
            [pltpu.SemaphoreType.DMA] * NBUF,
            [pltpu.SemaphoreType.DMA] * NBUF,
            pltpu.VMEM_SHARED((NPAD, HID2), jnp.float32),
        ],
    )
    def _sc_aggregate(h_hbm, src_hbm, dst_hbm, out_hbm,
                      src_v, dst_v, bufs, zbuf, gsems, ssems, acc):
        c = lax.axis_index("c")
        s = lax.axis_index("s")
        w = c * NS + s
        row0 = s * RPS
        pltpu.sync_copy(src_hbm.at[w], src_v)
        pltpu.sync_copy(dst_hbm.at[s if dst_by_s else w], dst_v)
        for b in range(NBUF):
            pltpu.async_copy(h_hbm.at[src_v.at[b]], bufs[b], gsems[b])

        @pl.loop(0, CHUNK)
        def _(r):
            for k in range(0, HID2, 16):
                zbuf[r, pl.ds(k, 16)] = jnp.zeros((16,), jnp.float32)

        _zero_rows(zbuf, acc, row0, HID2)
        plsc.subcore_barrier()

        @pl.loop(0, nch - NBUF, step=NBUF)
        def _(j):
            for b in range(NBUF):
                pltpu.make_async_copy(
                    h_hbm.at[src_v.at[j + b]], bufs[b], gsems[b]).wait()
                pltpu.async_copy(
                    bufs[b], acc.at[dst_v.at[j + b]], ssems[b], add=True)
            for b in range(NBUF):
                pltpu.make_async_copy(
                    bufs[b], acc.at[dst_v.at[j + b]], ssems[b]).wait()
                pltpu.async_copy(
                    h_hbm.at[src_v.at[j + b + NBUF]], bufs[b], gsems[b])

        for b in range(NBUF):
            j = nch - NBUF + b
            pltpu.make_async_copy(
                h_hbm.at[src_v.at[j]], bufs[b], gsems[b]).wait()
            pltpu.async_copy(
                bufs[b], acc.at[dst_v.at[j]], ssems[b], add=True)
        for b in range(NBUF):
            j = nch - NBUF + b
            pltpu.make_async_copy(
                bufs[b], acc.at[dst_v.at[j]], ssems[b]).wait()

        plsc.subcore_barrier()
        pltpu.sync_copy(acc.at[pl.ds(row0, RPS)],
                        out_hbm.at[pl.ds(c * NPAD + row0, RPS)])

    return _sc_aggregate


_sc_agg_l1 = _make_sc_aggregate(NCH1, NC * NPAD, True)
_sc_agg_l2 = _make_sc_aggregate(NCH2, NPAD, False)


def _tc_mm_norm(x, w, degp):
    def body(x_ref, w_ref, dp_ref, h1_ref, dis_ref, hs_ref):
        h1 = jnp.dot(x_ref[...], w_ref[...],
                     preferred_element_type=jnp.float32)
        deg = dp_ref[0:NPAD, 0:1] + dp_ref[NPAD:, 0:1] + 1.0
        dis = lax.rsqrt(deg)
        dis_ref[...] = dis
        h1_ref[pl.ds(0, N), :] = h1
        hs = h1 * dis[:N]
        hs_ref[pl.ds(0, N), :] = hs[:, :HID2]
        hs_ref[pl.ds(NPAD, N), :] = hs[:, HID2:]

    return pl.pallas_call(
        body,
        out_shape=(
            jax.ShapeDtypeStruct((NPAD, IN_DIM), jnp.float32),
            jax.ShapeDtypeStruct((NPAD, 1), jnp.float32),
            jax.ShapeDtypeStruct((NC * NPAD, HID2), jnp.float32),
        ),
    )(x, w, degp)


def _tc_layer(agg, h, dis, b, w):
    def body(a_ref, h_ref, dis_ref, b_ref, w_ref, h2_ref, h2s_ref):
        dis = dis_ref[0:N]
        agg = jnp.concatenate(
            [a_ref[0:N], a_ref[NPAD:NPAD + N]], axis=1)
        pre = agg * dis + h_ref[0:N] * (dis * dis) + b_ref[...]
        o1 = jnp.maximum(pre, 0.0)
        h2 = jnp.dot(o1, w_ref[...], preferred_element_type=jnp.float32)
        h2_ref[pl.ds(0, N), :] = h2
        h2s_ref[pl.ds(0, N), :] = h2 * dis

    d2 = w.shape[1]
    return pl.pallas_call(
        body,
        out_shape=(
            jax.ShapeDtypeStruct((NPAD, d2), jnp.float32),
            jax.ShapeDtypeStruct((NPAD, d2), jnp.float32),
        ),
    )(agg, h, dis, b, w)


def _tc_final(agg, h, dis, b, wf, bf):
    def body(a_ref, h_ref, dis_ref, b_ref, w_ref, bf_ref, o_ref):
        dis = dis_ref[0:N]
        pre = ((a_ref[0:N] + a_ref[NPAD:NPAD + N]) * dis
               + h_ref[0:N] * (dis * dis) + b_ref[...])
        o2 = jnp.maximum(pre, 0.0)
        o_ref[...] = jnp.dot(o2, w_ref[...],
                             preferred_element_type=jnp.float32) + bf_ref[...]

    return pl.pallas_call(
        body,
        out_shape=jax.ShapeDtypeStruct((N, N_CLASSES), jnp.float32),
    )(agg, h, dis, b, wf, bf)


def kernel(x, edge_index, W1, b1, W2, b2, Wf, bf):
    src = edge_index[0].astype(jnp.int32)
    dst = edge_index[1].astype(jnp.int32)
    pad = DUMMY + jnp.arange(E_PAD - E, dtype=jnp.int32) % (NPAD - N)
    src_p = jnp.concatenate([src, pad])
    dst_p = jnp.concatenate([dst, pad])

    src16 = src_p.reshape(NS, NCH1, CHUNK)
    dst16 = dst_p.reshape(NS, NCH1, CHUNK)
    src_l1 = jnp.concatenate([src16, src16 + NPAD], axis=0)

    src_l2 = src_p.reshape(NW, NCH2, CHUNK)
    dst_l2 = dst_p.reshape(NW, NCH2, CHUNK)

    degp = _sc_degree(dst_l2)
    h1, dis, h1s = _tc_mm_norm(x, W1, degp)

    agg1 = _sc_agg_l1(h1s, src_l1, dst16)
    h2, h2s = _tc_layer(agg1, h1, dis, b1.reshape(1, HID1), W2)

    agg2 = _sc_agg_l2(h2s, src_l2, dst_l2)
    return _tc_final(agg2, h2, dis, b2.reshape(1, HID2), Wf,
                     bf.reshape(1, N_CLASSES))


# speedup vs baseline: 34.7988x; 1.0409x over previous
import functools

import jax
import jax.numpy as jnp
from jax import lax
from jax.experimental import pallas as pl
from jax.experimental.pallas import tpu as pltpu
from jax.experimental.pallas import tpu_sc as plsc

N = 10000
E = 320000
IN_DIM = 128
HID1 = 128
HID2 = 64
N_CLASSES = 32

NC = 2
NS = 16
NW = NC * NS
CHUNK = 128
NBUF = 4
E_PAD = 327680
NCH1 = E_PAD // (NS * CHUNK)
NCH2 = E_PAD // (NW * CHUNK)
DUMMY = N
NPAD = 10112
RPS = NPAD // NS

_mesh = plsc.VectorSubcoreMesh(core_axis_name="c", subcore_axis_name="s")
_sc_params = pltpu.CompilerParams(use_tc_tiling_on_sc=False)


def _zero_rows(zbuf, acc, row0, width):
    del width
    for q in range(RPS // CHUNK):
        pltpu.sync_copy(zbuf, acc.at[pl.ds(row0 + q * CHUNK, CHUNK)])
    rem = RPS - (RPS // CHUNK) * CHUNK
    if rem:
        pltpu.sync_copy(zbuf.at[pl.ds(0, rem)],
                        acc.at[pl.ds(row0 + (RPS // CHUNK) * CHUNK, rem)])


@functools.partial(
    pl.kernel,
    out_type=jax.ShapeDtypeStruct((NC * NPAD, 16), jnp.float32),
    mesh=_mesh,
    compiler_params=_sc_params,
    scratch_types=[
        pltpu.VMEM((NCH2, CHUNK), jnp.int32),
        pltpu.VMEM((CHUNK, 16), jnp.float32),
        pltpu.VMEM((CHUNK, 16), jnp.float32),
        pltpu.VMEM_SHARED((NPAD, 16), jnp.float32),
    ],
)
def _sc_degree(dst_hbm, out_hbm, idx_v, ones_v, zbuf, acc):
    c = lax.axis_index("c")
    s = lax.axis_index("s")
    w = c * NS + s
    row0 = s * RPS
    pltpu.sync_copy(dst_hbm.at[w], idx_v)

    @pl.loop(0, CHUNK)
    def _(r):
        ones_v[r, pl.ds(0, 16)] = jnp.full((16,), 1.0, jnp.float32)
        zbuf[r, pl.ds(0, 16)] = jnp.zeros((16,), jnp.float32)

    _zero_rows(zbuf, acc, row0, 16)
    plsc.subcore_barrier()

    @pl.loop(0, NCH2)
    def _(j):
        pltpu.sync_copy(ones_v, acc.at[idx_v.at[j]], add=True)

    plsc.subcore_barrier()
    pltpu.sync_copy(acc.at[pl.ds(row0, RPS)],
                    out_hbm.at[pl.ds(c * NPAD + row0, RPS)])


def _make_sc_aggregate(nch, rows_in, dst_by_s):
    @functools.partial(
        pl.kernel,
        out_type=jax.ShapeDtypeStruct((NC * NPAD, HID2), jnp.float32),
        mesh=_mesh,
        compiler_params=_sc_params,
        scratch_types=[
            pltpu.VMEM((nch, CHUNK), jnp.int32),
            pltpu.VMEM((nch, CHUNK), jnp.int32),
            [pltpu.VMEM((CHUNK, HID2), jnp.float32)] * NBUF,
            pltpu.VMEM((CHUNK, HID2), jnp.float32),
            [pltpu.SemaphoreType.DMA] * NBUF,
            [pltpu.SemaphoreType.DMA] * NBUF,
            pltpu.VMEM_SHARED((NPAD, HID2), jnp.float32),
        ],
    )
    def _sc_aggregate(h_hbm, src_hbm, dst_hbm, out_hbm,
                      src_v, dst_v, bufs, zbuf, gsems, ssems, acc):
        c = lax.axis_index("c")
        s = lax.axis_index("s")
        w = c * NS + s
        row0 = s * RPS
        pltpu.sync_copy(src_hbm.at[w], src_v)
        pltpu.sync_copy(dst_hbm.at[s if dst_by_s else w], dst_v)
        for b in range(NBUF):
            pltpu.async_copy(h_hbm.at[src_v.at[b]], bufs[b], gsems[b])

        @pl.loop(0, CHUNK)
        def _(r):
            for k in range(0, HID2, 16):
                zbuf[r, pl.ds(k, 16)] = jnp.zeros((16,), jnp.float32)

        _zero_rows(zbuf, acc, row0, HID2)
        plsc.subcore_barrier()

        @pl.loop(0, nch - NBUF, step=NBUF)
        def _(j):
            for b in range(NBUF):
                pltpu.make_async_copy(
                    h_hbm.at[src_v.at[j + b]], bufs[b], gsems[b]).wait()
                pltpu.async_copy(
                    bufs[b], acc.at[dst_v.at[j + b]], ssems[b], add=True)
            for b in range(NBUF):
                pltpu.make_async_copy(
                    bufs[b], acc.at[dst_v.at[j + b]], ssems[b]).wait()
                pltpu.async_copy(
                    h_hbm.at[src_v.at[j + b + NBUF]], bufs[b], gsems[b])

        for b in range(NBUF):
            j = nch - NBUF + b
            pltpu.make_async_copy(
                h_hbm.at[src_v.at[j]], bufs[b], gsems[b]).wait()
            pltpu.async_copy(
                bufs[b], acc.at[dst_v.at[j]], ssems[b], add=True)
        for b in range(NBUF):
            j = nch - NBUF + b
            pltpu.make_async_copy(
                bufs[b], acc.at[dst_v.at[j]], ssems[b]).wait()

        plsc.subcore_barrier()
        pltpu.sync_copy(acc.at[pl.ds(row0, RPS)],
                        out_hbm.at[pl.ds(c * NPAD + row0, RPS)])

    return _sc_aggregate


_sc_agg_l1 = _make_sc_aggregate(NCH1, NC * NPAD, True)
_sc_agg_l2 = _make_sc_aggregate(NCH2, NPAD, False)


def _tc_mm_norm(x, w, degp):
    def body(x_ref, w_ref, dp_ref, h1_ref, dis_ref, hs_ref):
        h1 = jnp.dot(x_ref[...], w_ref[...],
                     preferred_element_type=jnp.float32)
        deg = dp_ref[0:NPAD, 0:1] + dp_ref[NPAD:, 0:1] + 1.0
        dis = lax.rsqrt(deg)
        dis_ref[...] = dis
        h1_ref[pl.ds(0, N), :] = h1
        hs_ref[pl.ds(0, N), :] = h1 * dis[:N]

    return pl.pallas_call(
        body,
        out_shape=(
            jax.ShapeDtypeStruct((NPAD, IN_DIM), jnp.float32),
            jax.ShapeDtypeStruct((NPAD, 1), jnp.float32),
            jax.ShapeDtypeStruct((NPAD, IN_DIM), jnp.float32),
        ),
    )(x, w, degp)


def _tc_layer(agg, h, dis, b, w):
    def body(a_ref, h_ref, dis_ref, b_ref, w_ref, h2_ref, h2s_ref):
        dis = dis_ref[0:N]
        agg = jnp.concatenate(
            [a_ref[0:N], a_ref[NPAD:NPAD + N]], axis=1)
        pre = agg * dis + h_ref[0:N] * (dis * dis) + b_ref[...]
        o1 = jnp.maximum(pre, 0.0)
        h2 = jnp.dot(o1, w_ref[...], preferred_element_type=jnp.float32)
        h2_ref[pl.ds(0, N), :] = h2
        h2s_ref[pl.ds(0, N), :] = h2 * dis

    d2 = w.shape[1]
    return pl.pallas_call(
        body,
        out_shape=(
            jax.ShapeDtypeStruct((NPAD, d2), jnp.float32),
            jax.ShapeDtypeStruct((NPAD, d2), jnp.float32),
        ),
    )(agg, h, dis, b, w)


def _tc_final(agg, h, dis, b, wf, bf):
    def body(a_ref, h_ref, dis_ref, b_ref, w_ref, bf_ref, o_ref):
        dis = dis_ref[0:N]
        pre = ((a_ref[0:N] + a_ref[NPAD:NPAD + N]) * dis
               + h_ref[0:N] * (dis * dis) + b_ref[...])
        o2 = jnp.maximum(pre, 0.0)
        o_ref[...] = jnp.dot(o2, w_ref[...],
                             preferred_element_type=jnp.float32) + bf_ref[...]

    return pl.pallas_call(
        body,
        out_shape=jax.ShapeDtypeStruct((N, N_CLASSES), jnp.float32),
    )(agg, h, dis, b, wf, bf)


def kernel(x, edge_index, W1, b1, W2, b2, Wf, bf):
    src = edge_index[0].astype(jnp.int32)
    dst = edge_index[1].astype(jnp.int32)
    pad = DUMMY + jnp.arange(E_PAD - E, dtype=jnp.int32) % (NPAD - N)
    src_p = jnp.concatenate([src, pad])
    dst_p = jnp.concatenate([dst, pad])

    src16 = src_p.reshape(NS, NCH1, CHUNK)
    dst16 = dst_p.reshape(NS, NCH1, CHUNK)
    src_l1 = jnp.concatenate([src16 * 2, src16 * 2 + 1], axis=0)

    src_l2 = src_p.reshape(NW, NCH2, CHUNK)
    dst_l2 = dst_p.reshape(NW, NCH2, CHUNK)

    degp = _sc_degree(dst_l2)
    h1, dis, h1s = _tc_mm_norm(x, W1, degp)

    agg1 = _sc_agg_l1(h1s.reshape(NC * NPAD, HID2), src_l1, dst16)
    h2, h2s = _tc_layer(agg1, h1, dis, b1.reshape(1, HID1), W2)

    agg2 = _sc_agg_l2(h2s, src_l2, dst_l2)
    return _tc_final(agg2, h2, dis, b2.reshape(1, HID2), Wf,
                     bf.reshape(1, N_CLASSES))
